# Initial kernel scaffold; baseline (speedup 1.0000x reference)
import functools
import jax
import jax.numpy as jnp
from jax.experimental import pallas as pl
from jax.experimental.pallas import tpu as pltpu

N = 10000
C_IN = 32
C_OUT = 32
HID = 64
K = 64
Q_TILE = 250
GRID = N // Q_TILE


def _ln(x, g, b, eps=1e-5):
    m = jnp.mean(x, axis=-1, keepdims=True)
    v = jnp.mean((x - m) * (x - m), axis=-1, keepdims=True)
    return (x - m) * jax.lax.rsqrt(v + eps) * g + b


def _gelu(x):
    return x * 0.5 * (1.0 + jax.lax.erf(x * 0.7071067811865476))


def _edge_body(feats_ref, verts_ref, nbrF_ref, nbrV_ref,
               w1a_ref, w1b_ref, w1c_ref, b1_ref, g1_ref, bt1_ref,
               w2_ref, b2_ref,
               wsa_ref, wsb_ref, wsc_ref, bs_ref, g2_ref, bt2_ref,
               ow1_ref, ob1_ref, og1_ref, obt1_ref,
               ow2_ref, ob2_ref, og2_ref, obt2_ref,
               out_ref):
    E = Q_TILE * K
    self_f = feats_ref[...]
    qv = verts_ref[...]
    nbrF = nbrF_ref[...]
    nbrV = nbrV_ref[...]

    selfe = jnp.broadcast_to(self_f[:, None, :], (Q_TILE, K, C_IN)).reshape(E, C_IN)
    rel = (nbrV.reshape(Q_TILE, K, 3) - qv[:, None, :]).reshape(E, 3)

    h = (jnp.dot(selfe, w1a_ref[...], preferred_element_type=jnp.float32)
         + jnp.dot(nbrF, w1b_ref[...], preferred_element_type=jnp.float32)
         + jnp.dot(rel, w1c_ref[...], preferred_element_type=jnp.float32)
         + b1_ref[...])
    h = _gelu(_ln(h, g1_ref[...], bt1_ref[...]))
    h2 = jnp.dot(h, w2_ref[...], preferred_element_type=jnp.float32) + b2_ref[...]
    sc = (jnp.dot(selfe, wsa_ref[...], preferred_element_type=jnp.float32)
          + jnp.dot(nbrF, wsb_ref[...], preferred_element_type=jnp.float32)
          + jnp.dot(rel, wsc_ref[...], preferred_element_type=jnp.float32)
          + bs_ref[...])
    e = _gelu(_ln(h2 + sc, g2_ref[...], bt2_ref[...]))

    red = jnp.mean(e.reshape(Q_TILE, K, C_OUT), axis=1)

    oh = jnp.dot(red, ow1_ref[...], preferred_element_type=jnp.float32) + ob1_ref[...]
    oh = _gelu(_ln(oh, og1_ref[...], obt1_ref[...]))
    oh2 = jnp.dot(oh, ow2_ref[...], preferred_element_type=jnp.float32) + ob2_ref[...]
    out_ref[...] = _gelu(_ln(oh2 + red, og2_ref[...], obt2_ref[...]))


def _rep(shape):
    return pl.BlockSpec(shape, lambda i: (0,) * len(shape))


def _edge_pallas(feats, verts, nbrF, nbrV, weights):
    E = Q_TILE * K
    in_specs = [
        pl.BlockSpec((Q_TILE, C_IN), lambda i: (i, 0)),
        pl.BlockSpec((Q_TILE, 3), lambda i: (i, 0)),
        pl.BlockSpec((E, C_IN), lambda i: (i, 0)),
        pl.BlockSpec((E, 3), lambda i: (i, 0)),
    ] + [_rep(w.shape) for w in weights]
    return pl.pallas_call(
        _edge_body,
        grid=(GRID,),
        in_specs=in_specs,
        out_specs=pl.BlockSpec((Q_TILE, C_OUT), lambda i: (i, 0)),
        out_shape=jax.ShapeDtypeStruct((N, C_OUT), jnp.float32),
    )(feats, verts, nbrF, nbrV, *weights)


def _knn_idx(in_v, k, chunk=2000):
    qs = in_v.reshape(N // chunk, chunk, 3)

    def body(q):
        d = jnp.sum((q[:, None, :] - in_v[None, :, :]) ** 2, axis=-1)
        _, idx = jax.lax.top_k(-d, k)
        return idx

    idx = jax.lax.map(body, qs)
    return idx.reshape(N * k)


def kernel(vertices, features, e_w1, e_b1, e_g1, e_bt1, e_w2, e_b2, e_ws,
           e_bs, e_g2, e_bt2, o_w1, o_b1, o_g1, o_bt1, o_w2, o_b2, o_g2,
           o_bt2):
    B = vertices.shape[0]
    in_v = vertices.reshape(N, 3)
    feats = features.reshape(N, C_IN)

    idx = _knn_idx(in_v, K)
    nbrF = feats[idx]
    nbrV = in_v[idx]

    weights = (
        e_w1[:C_IN], e_w1[C_IN:2 * C_IN], e_w1[2 * C_IN:], e_b1, e_g1, e_bt1,
        e_w2, e_b2,
        e_ws[:C_IN], e_ws[C_IN:2 * C_IN], e_ws[2 * C_IN:], e_bs, e_g2, e_bt2,
        o_w1, o_b1, o_g1, o_bt1, o_w2, o_b2, o_g2, o_bt2,
    )
    out = _edge_pallas(feats, in_v, nbrF, nbrV, weights)
    return out.reshape(B, N, C_OUT)


# speedup vs baseline: 1.0142x; 1.0142x over previous
import functools
import jax
import jax.numpy as jnp
from jax.experimental import pallas as pl
from jax.experimental.pallas import tpu as pltpu

N = 10000
C_IN = 32
C_OUT = 32
HID = 64
K = 64
Q_TILE = 80
GRID = N // Q_TILE


def _ln(x, g, b, eps=1e-5):
    m = jnp.mean(x, axis=-1, keepdims=True)
    v = jnp.mean((x - m) * (x - m), axis=-1, keepdims=True)
    return (x - m) * jax.lax.rsqrt(v + eps) * g + b


def _gelu(x):
    return x * 0.5 * (1.0 + jax.lax.erf(x * 0.7071067811865476))


def _edge_body(feats_ref, verts_ref, nbrF_ref, nbrV_ref,
               w1a_ref, w1b_ref, w1c_ref, b1_ref, g1_ref, bt1_ref,
               w2_ref, b2_ref,
               wsa_ref, wsb_ref, wsc_ref, bs_ref, g2_ref, bt2_ref,
               ow1_ref, ob1_ref, og1_ref, obt1_ref,
               ow2_ref, ob2_ref, og2_ref, obt2_ref,
               out_ref):
    E = Q_TILE * K
    self_f = feats_ref[...]
    qv = verts_ref[...]
    nbrF = nbrF_ref[...]
    nbrV = nbrV_ref[...]

    selfe = jnp.broadcast_to(self_f[:, None, :], (Q_TILE, K, C_IN)).reshape(E, C_IN)
    rel = (nbrV.reshape(Q_TILE, K, 3) - qv[:, None, :]).reshape(E, 3)

    h = (jnp.dot(selfe, w1a_ref[...], preferred_element_type=jnp.float32)
         + jnp.dot(nbrF, w1b_ref[...], preferred_element_type=jnp.float32)
         + jnp.dot(rel, w1c_ref[...], preferred_element_type=jnp.float32)
         + b1_ref[...])
    h = _gelu(_ln(h, g1_ref[...], bt1_ref[...]))
    h2 = jnp.dot(h, w2_ref[...], preferred_element_type=jnp.float32) + b2_ref[...]
    sc = (jnp.dot(selfe, wsa_ref[...], preferred_element_type=jnp.float32)
          + jnp.dot(nbrF, wsb_ref[...], preferred_element_type=jnp.float32)
          + jnp.dot(rel, wsc_ref[...], preferred_element_type=jnp.float32)
          + bs_ref[...])
    e = _gelu(_ln(h2 + sc, g2_ref[...], bt2_ref[...]))

    red = jnp.mean(e.reshape(Q_TILE, K, C_OUT), axis=1)

    oh = jnp.dot(red, ow1_ref[...], preferred_element_type=jnp.float32) + ob1_ref[...]
    oh = _gelu(_ln(oh, og1_ref[...], obt1_ref[...]))
    oh2 = jnp.dot(oh, ow2_ref[...], preferred_element_type=jnp.float32) + ob2_ref[...]
    out_ref[...] = _gelu(_ln(oh2 + red, og2_ref[...], obt2_ref[...]))


def _rep(shape):
    return pl.BlockSpec(shape, lambda i: (0,) * len(shape))


def _edge_pallas(feats, verts, nbrF, nbrV, weights):
    E = Q_TILE * K
    in_specs = [
        pl.BlockSpec((Q_TILE, C_IN), lambda i: (i, 0)),
        pl.BlockSpec((Q_TILE, 3), lambda i: (i, 0)),
        pl.BlockSpec((E, C_IN), lambda i: (i, 0)),
        pl.BlockSpec((E, 3), lambda i: (i, 0)),
    ] + [_rep(w.shape) for w in weights]
    return pl.pallas_call(
        _edge_body,
        grid=(GRID,),
        in_specs=in_specs,
        out_specs=pl.BlockSpec((Q_TILE, C_OUT), lambda i: (i, 0)),
        out_shape=jax.ShapeDtypeStruct((N, C_OUT), jnp.float32),
    )(feats, verts, nbrF, nbrV, *weights)


def _knn_idx(in_v, k, chunk=2000):
    qs = in_v.reshape(N // chunk, chunk, 3)

    def body(q):
        d = jnp.sum((q[:, None, :] - in_v[None, :, :]) ** 2, axis=-1)
        _, idx = jax.lax.top_k(-d, k)
        return idx

    idx = jax.lax.map(body, qs)
    return idx.reshape(N * k)


def kernel(vertices, features, e_w1, e_b1, e_g1, e_bt1, e_w2, e_b2, e_ws,
           e_bs, e_g2, e_bt2, o_w1, o_b1, o_g1, o_bt1, o_w2, o_b2, o_g2,
           o_bt2):
    B = vertices.shape[0]
    in_v = vertices.reshape(N, 3)
    feats = features.reshape(N, C_IN)

    idx = _knn_idx(in_v, K)
    nbrF = feats[idx]
    nbrV = in_v[idx]

    weights = (
        e_w1[:C_IN], e_w1[C_IN:2 * C_IN], e_w1[2 * C_IN:], e_b1, e_g1, e_bt1,
        e_w2, e_b2,
        e_ws[:C_IN], e_ws[C_IN:2 * C_IN], e_ws[2 * C_IN:], e_bs, e_g2, e_bt2,
        o_w1, o_b1, o_g1, o_bt1, o_w2, o_b2, o_g2, o_bt2,
    )
    out = _edge_pallas(feats, in_v, nbrF, nbrV, weights)
    return out.reshape(B, N, C_OUT)
